# baseline (speedup 1.0000x reference)
import functools

import jax
import jax.numpy as jnp
from jax import lax
from jax.experimental import pallas as pl
from jax.experimental.pallas import tpu as pltpu
from jax.experimental.pallas import tpu_sc as plsc

_NC, _NS, _L = 2, 16, 16
_B, _C, _H, _W = 16, 64, 128, 128
_HW = _H * _W
_N = 128 * 17
_NV = _N // _L
_CPT = _C // 2


def _tr_body(t_ref, o_ref):
    o_ref[0] = t_ref[0].T


def _transpose_target(target2):
    b, n, c = target2.shape
    return pl.pallas_call(
        _tr_body,
        grid=(b,),
        in_specs=[pl.BlockSpec((1, n, c), lambda i: (i, 0, 0))],
        out_specs=pl.BlockSpec((1, c, n), lambda i: (i, 0, 0)),
        out_shape=jax.ShapeDtypeStruct((b, c, n), jnp.float32),
    )(target2)


def _sc_body(planes_hbm, tt_hbm, ind_hbm, mask_hbm, out_hbm,
             idx_v, m_v, plane_v, trow_v, red_v):
    cid = lax.axis_index("c")
    sid = lax.axis_index("s")
    g = cid * _NS + sid
    b = g // 2
    half = g % 2
    p0 = b * _C + half * _CPT

    pltpu.sync_copy(ind_hbm.at[b], idx_v)
    pltpu.sync_copy(mask_hbm.at[b], m_v)

    zeros = jnp.zeros((_L,), jnp.float32)

    def plane_step(j, acc):
        pltpu.sync_copy(planes_hbm.at[p0 + j], plane_v)
        pltpu.sync_copy(tt_hbm.at[p0 + j], trow_v)

        def inner(i, a):
            sl = pl.ds(i * _L, _L)
            idx = idx_v[sl]
            pred = plsc.load_gather(plane_v, [idx])
            t = trow_v[sl]
            m = m_v[sl]
            return a + jnp.abs(pred - t) * jnp.abs(m)

        return lax.fori_loop(0, _NV, inner, acc)

    acc = lax.fori_loop(0, _CPT, plane_step, zeros)

    def msum_step(i, a):
        return a + m_v[pl.ds(i * _L, _L)]

    msum = lax.fori_loop(0, _NV, msum_step, zeros)
    msum = msum * (half == 0).astype(jnp.float32)

    red_v[0, :] = acc
    red_v[1, :] = msum
    pltpu.sync_copy(red_v, out_hbm.at[cid, sid])


@functools.cache
def _sc_kernel():
    return functools.partial(
        pl.kernel,
        out_type=jax.ShapeDtypeStruct((_NC, _NS, 2, _L), jnp.float32),
        mesh=plsc.VectorSubcoreMesh(
            core_axis_name="c", subcore_axis_name="s",
            num_cores=_NC, num_subcores=_NS),
        compiler_params=pltpu.CompilerParams(needs_layout_passes=False),
        scratch_types=[
            pltpu.VMEM((_N,), jnp.int32),
            pltpu.VMEM((_N,), jnp.float32),
            pltpu.VMEM((_HW,), jnp.float32),
            pltpu.VMEM((_N,), jnp.float32),
            pltpu.VMEM((2, _L), jnp.float32),
        ],
    )(_sc_body)


def kernel(output, target, ind, ind_mask):
    b, C, H, W = output.shape
    planes = output.reshape(b * C, H * W)
    target2 = target.reshape(b, _N, C)
    tt = _transpose_target(target2).reshape(b * C, _N)
    parts = _sc_kernel()(planes, tt,
                         ind.reshape(b, _N), ind_mask.reshape(b, _N))
    abs_sum = jnp.sum(parts[:, :, 0, :])
    mask_sum = jnp.sum(parts[:, :, 1, :])
    return abs_sum / (C * mask_sum + 0.0001)


# speedup vs baseline: 1.1771x; 1.1771x over previous
import functools

import jax
import jax.numpy as jnp
from jax import lax
from jax.experimental import pallas as pl
from jax.experimental.pallas import tpu as pltpu
from jax.experimental.pallas import tpu_sc as plsc

_NC, _NS, _L = 2, 16, 16
_B, _C, _H, _W = 16, 64, 128, 128
_HW = _H * _W
_N = 128 * 17
_NV = _N // _L
_CPT = _C // 2


def _tr_body(t_ref, o_ref):
    o_ref[0] = t_ref[0].T


def _transpose_target(target2):
    b, n, c = target2.shape
    return pl.pallas_call(
        _tr_body,
        grid=(b,),
        in_specs=[pl.BlockSpec((1, n, c), lambda i: (i, 0, 0))],
        out_specs=pl.BlockSpec((1, c, n), lambda i: (i, 0, 0)),
        out_shape=jax.ShapeDtypeStruct((b, c, n), jnp.float32),
    )(target2)


def _sc_body(planes_hbm, tt_hbm, ind_hbm, mask_hbm, out_hbm,
             idx_v, m_v, plane_v, trow_v, red_v):
    cid = lax.axis_index("c")
    sid = lax.axis_index("s")
    g = cid * _NS + sid
    b = g // 2
    half = g % 2
    c0 = half * _CPT

    pltpu.sync_copy(ind_hbm.at[b], idx_v)
    pltpu.sync_copy(mask_hbm.at[b], m_v)

    zeros = jnp.zeros((_L,), jnp.float32)

    def plane_step(j, acc):
        pltpu.sync_copy(planes_hbm.at[b, c0 + j], plane_v)
        pltpu.sync_copy(tt_hbm.at[b * _C + c0 + j], trow_v)

        def inner(i, a):
            sl = pl.ds(i * _L, _L)
            idx = idx_v[sl]
            ih = lax.shift_right_logical(idx, 7)
            iw = lax.bitwise_and(idx, 127)
            pred = plsc.load_gather(plane_v, [ih, iw])
            t = trow_v[sl]
            m = m_v[sl]
            return a + jnp.abs(pred - t) * jnp.abs(m)

        return lax.fori_loop(0, _NV, inner, acc)

    acc = lax.fori_loop(0, _CPT, plane_step, zeros)

    def msum_step(i, a):
        return a + m_v[pl.ds(i * _L, _L)]

    msum = lax.fori_loop(0, _NV, msum_step, zeros)
    msum = msum * (half == 0).astype(jnp.float32)

    red_v[0, :] = acc
    red_v[1, :] = msum
    pltpu.sync_copy(red_v, out_hbm.at[cid, sid])


@functools.cache
def _sc_kernel():
    return functools.partial(
        pl.kernel,
        out_type=jax.ShapeDtypeStruct((_NC, _NS, 2, _L), jnp.float32),
        mesh=plsc.VectorSubcoreMesh(
            core_axis_name="c", subcore_axis_name="s",
            num_cores=_NC, num_subcores=_NS),
        compiler_params=pltpu.CompilerParams(
            needs_layout_passes=False, use_tc_tiling_on_sc=False),
        scratch_types=[
            pltpu.VMEM((_N,), jnp.int32),
            pltpu.VMEM((_N,), jnp.float32),
            pltpu.VMEM((_H, _W), jnp.float32),
            pltpu.VMEM((_N,), jnp.float32),
            pltpu.VMEM((2, _L), jnp.float32),
        ],
    )(_sc_body)


def kernel(output, target, ind, ind_mask):
    b, C, H, W = output.shape
    target2 = target.reshape(b, _N, C)
    tt = _transpose_target(target2).reshape(b * C, _N)
    parts = _sc_kernel()(output, tt,
                         ind.reshape(b, _N), ind_mask.reshape(b, _N))
    abs_sum = jnp.sum(parts[:, :, 0, :])
    mask_sum = jnp.sum(parts[:, :, 1, :])
    return abs_sum / (C * mask_sum + 0.0001)


# speedup vs baseline: 1.7078x; 1.4509x over previous
import functools

import jax
import jax.numpy as jnp
from jax import lax
from jax.experimental import pallas as pl
from jax.experimental.pallas import tpu as pltpu
from jax.experimental.pallas import tpu_sc as plsc

_NC, _NS, _L = 2, 16, 16
_B, _C, _H, _W = 16, 64, 128, 128
_MO, _MP = 128, 17
_N = _MO * _MP
_NV = _N // _L
_CPT = _C // 2


def _sc_body(planes_hbm, tgt_hbm, ind_hbm, mask_hbm, out_hbm,
             idx_v, m_v, tv_v, pl0_v, pl1_v, red_v, sem0, sem1):
    cid = lax.axis_index("c")
    sid = lax.axis_index("s")
    g = cid * _NS + sid
    b = g // 2
    half = g % 2
    c0 = half * _CPT

    cp0 = pltpu.async_copy(planes_hbm.at[b, c0], pl0_v, sem0)
    pltpu.sync_copy(tgt_hbm.at[b, :, :, pl.ds(c0, _CPT)], tv_v)
    pltpu.sync_copy(ind_hbm.at[b], idx_v)
    pltpu.sync_copy(mask_hbm.at[b], m_v)

    zeros = jnp.zeros((_L,), jnp.float32)
    izeros = jnp.zeros((_L,), jnp.int32)
    iota = lax.broadcasted_iota(jnp.int32, (_L,), 0)

    def plane_acc(plane_ref, j, acc):
        jv = jnp.full((_L,), j, jnp.int32)

        def inner(i, carry):
            a, io, ip = carry
            sl = pl.ds(i * _L, _L)
            idx = idx_v[sl]
            ih = lax.shift_right_logical(idx, 7)
            iw = lax.bitwise_and(idx, 127)
            pred = plsc.load_gather(plane_ref, [ih, iw])
            t = plsc.load_gather(tv_v, [io, ip, jv])
            m = m_v[sl]
            a = a + jnp.abs(pred - t) * jnp.abs(m)
            ge = (ip >= 1).astype(jnp.int32)
            io = io + ge
            ip = ip + 16 - 17 * ge
            return (a, io, ip)

        acc, _, _ = lax.fori_loop(0, _NV, inner, (acc, izeros, iota))
        return acc

    def step(jj, acc):
        j0 = 2 * jj
        pltpu.make_async_copy(planes_hbm.at[b, c0], pl0_v, sem0).wait()
        pltpu.async_copy(planes_hbm.at[b, c0 + j0 + 1], pl1_v, sem1)
        acc = plane_acc(pl0_v, j0, acc)
        pltpu.make_async_copy(planes_hbm.at[b, c0], pl1_v, sem1).wait()

        @pl.when(jj < _CPT // 2 - 1)
        def _():
            pltpu.async_copy(planes_hbm.at[b, c0 + j0 + 2], pl0_v, sem0)

        return plane_acc(pl1_v, j0 + 1, acc)

    acc = lax.fori_loop(0, _CPT // 2, step, zeros)

    def msum_step(i, a):
        return a + m_v[pl.ds(i * _L, _L)]

    msum = lax.fori_loop(0, _NV, msum_step, zeros)
    msum = msum * (half == 0).astype(jnp.float32)

    red_v[0, :] = acc
    red_v[1, :] = msum
    pltpu.sync_copy(red_v, out_hbm.at[cid, sid])


@functools.cache
def _sc_kernel():
    return functools.partial(
        pl.kernel,
        out_type=jax.ShapeDtypeStruct((_NC, _NS, 2, _L), jnp.float32),
        mesh=plsc.VectorSubcoreMesh(
            core_axis_name="c", subcore_axis_name="s",
            num_cores=_NC, num_subcores=_NS),
        compiler_params=pltpu.CompilerParams(
            needs_layout_passes=False, use_tc_tiling_on_sc=False),
        scratch_types=[
            pltpu.VMEM((_N,), jnp.int32),
            pltpu.VMEM((_N,), jnp.float32),
            pltpu.VMEM((_MO, _MP, _CPT), jnp.float32),
            pltpu.VMEM((_H, _W), jnp.float32),
            pltpu.VMEM((_H, _W), jnp.float32),
            pltpu.VMEM((2, _L), jnp.float32),
            pltpu.SemaphoreType.DMA,
            pltpu.SemaphoreType.DMA,
        ],
    )(_sc_body)


def kernel(output, target, ind, ind_mask):
    b, C, H, W = output.shape
    parts = _sc_kernel()(output, target,
                         ind.reshape(b, _N), ind_mask.reshape(b, _N))
    abs_sum = jnp.sum(parts[:, :, 0, :])
    mask_sum = jnp.sum(parts[:, :, 1, :])
    return abs_sum / (C * mask_sum + 0.0001)


# speedup vs baseline: 1.8657x; 1.0925x over previous
import functools

import jax
import jax.numpy as jnp
from jax import lax
from jax.experimental import pallas as pl
from jax.experimental.pallas import tpu as pltpu
from jax.experimental.pallas import tpu_sc as plsc

_NC, _NS, _L = 2, 16, 16
_B, _C, _H, _W = 16, 64, 128, 128
_MO, _MP = 128, 17
_N = _MO * _MP
_NV = _N // _L
_CPT = _C // 2


def _sc_body(planes_hbm, tgt_hbm, ind_hbm, mask_hbm, out_hbm,
             idx_v, m_v, tv_v, pl0_v, pl1_v, red_v, sem0, sem1):
    cid = lax.axis_index("c")
    sid = lax.axis_index("s")
    g = cid * _NS + sid
    b = g // 2
    half = g % 2
    c0 = half * _CPT

    cp0 = pltpu.async_copy(planes_hbm.at[b, c0], pl0_v, sem0)
    pltpu.sync_copy(tgt_hbm.at[b, :, :, pl.ds(c0, _CPT)], tv_v)
    pltpu.sync_copy(ind_hbm.at[b], idx_v)
    pltpu.sync_copy(mask_hbm.at[b], m_v)

    zeros = jnp.zeros((_L,), jnp.float32)
    izeros = jnp.zeros((_L,), jnp.int32)
    iota = lax.broadcasted_iota(jnp.int32, (_L,), 0)

    def plane_acc(plane_ref, j, acc):
        jv = jnp.full((_L,), j, jnp.int32)

        def inner(i, carry):
            a, io, ip = carry
            sl = pl.ds(i * _L, _L)
            idx = idx_v[sl]
            ih = lax.shift_right_logical(idx, 7)
            iw = lax.bitwise_and(idx, 127)
            pred = plsc.load_gather(plane_ref, [ih, iw])
            t = plsc.load_gather(tv_v, [io, ip, jv])
            m = m_v[sl]
            a = a + jnp.abs(pred - t) * jnp.abs(m)
            ge = (ip >= 1).astype(jnp.int32)
            io = io + ge
            ip = ip + 16 - 17 * ge
            return (a, io, ip)

        acc, _, _ = lax.fori_loop(0, _NV, inner, (acc, izeros, iota),
                                  unroll=4)
        return acc

    def step(jj, acc):
        j0 = 2 * jj
        pltpu.make_async_copy(planes_hbm.at[b, c0], pl0_v, sem0).wait()
        pltpu.async_copy(planes_hbm.at[b, c0 + j0 + 1], pl1_v, sem1)
        acc = plane_acc(pl0_v, j0, acc)
        pltpu.make_async_copy(planes_hbm.at[b, c0], pl1_v, sem1).wait()

        @pl.when(jj < _CPT // 2 - 1)
        def _():
            pltpu.async_copy(planes_hbm.at[b, c0 + j0 + 2], pl0_v, sem0)

        return plane_acc(pl1_v, j0 + 1, acc)

    acc = lax.fori_loop(0, _CPT // 2, step, zeros)

    def msum_step(i, a):
        return a + m_v[pl.ds(i * _L, _L)]

    msum = lax.fori_loop(0, _NV, msum_step, zeros)
    msum = msum * (half == 0).astype(jnp.float32)

    red_v[0, :] = acc
    red_v[1, :] = msum
    pltpu.sync_copy(red_v, out_hbm.at[cid, sid])


@functools.cache
def _sc_kernel():
    return functools.partial(
        pl.kernel,
        out_type=jax.ShapeDtypeStruct((_NC, _NS, 2, _L), jnp.float32),
        mesh=plsc.VectorSubcoreMesh(
            core_axis_name="c", subcore_axis_name="s",
            num_cores=_NC, num_subcores=_NS),
        compiler_params=pltpu.CompilerParams(
            needs_layout_passes=False, use_tc_tiling_on_sc=False),
        scratch_types=[
            pltpu.VMEM((_N,), jnp.int32),
            pltpu.VMEM((_N,), jnp.float32),
            pltpu.VMEM((_MO, _MP, _CPT), jnp.float32),
            pltpu.VMEM((_H, _W), jnp.float32),
            pltpu.VMEM((_H, _W), jnp.float32),
            pltpu.VMEM((2, _L), jnp.float32),
            pltpu.SemaphoreType.DMA,
            pltpu.SemaphoreType.DMA,
        ],
    )(_sc_body)


def kernel(output, target, ind, ind_mask):
    b, C, H, W = output.shape
    parts = _sc_kernel()(output, target,
                         ind.reshape(b, _N), ind_mask.reshape(b, _N))
    abs_sum = jnp.sum(parts[:, :, 0, :])
    mask_sum = jnp.sum(parts[:, :, 1, :])
    return abs_sum / (C * mask_sum + 0.0001)


# speedup vs baseline: 1.8674x; 1.0009x over previous
import functools

import jax
import jax.numpy as jnp
from jax import lax
from jax.experimental import pallas as pl
from jax.experimental.pallas import tpu as pltpu
from jax.experimental.pallas import tpu_sc as plsc

_NC, _NS, _L = 2, 16, 16
_B, _C, _H, _W = 16, 64, 128, 128
_MO, _MP = 128, 17
_N = _MO * _MP
_NV = _N // _L
_CPT = _C // 2


def _sc_body(planes_hbm, tgt_hbm, ind_hbm, mask_hbm, out_hbm,
             idx_v, m_v, tv_v, pl0_v, pl1_v, red_v, sem0, sem1):
    cid = lax.axis_index("c")
    sid = lax.axis_index("s")
    g = cid * _NS + sid
    b = g // 2
    half = g % 2
    c0 = half * _CPT

    cp0 = pltpu.async_copy(planes_hbm.at[b, c0], pl0_v, sem0)
    pltpu.sync_copy(tgt_hbm.at[b, :, :, pl.ds(c0, _CPT)], tv_v)
    pltpu.sync_copy(ind_hbm.at[b], idx_v)
    pltpu.sync_copy(mask_hbm.at[b], m_v)

    zeros = jnp.zeros((_L,), jnp.float32)
    izeros = jnp.zeros((_L,), jnp.int32)
    iota = lax.broadcasted_iota(jnp.int32, (_L,), 0)

    def plane_acc(plane_ref, j, acc):
        jv = jnp.full((_L,), j, jnp.int32)

        def inner(i, carry):
            a, io, ip = carry
            sl = pl.ds(i * _L, _L)
            idx = idx_v[sl]
            ih = lax.shift_right_logical(idx, 7)
            iw = lax.bitwise_and(idx, 127)
            pred = plsc.load_gather(plane_ref, [ih, iw])
            t = plsc.load_gather(tv_v, [io, ip, jv])
            m = m_v[sl]
            a = a + jnp.abs(pred - t) * jnp.abs(m)
            ge = (ip >= 1).astype(jnp.int32)
            io = io + ge
            ip = ip + 16 - 17 * ge
            return (a, io, ip)

        acc, _, _ = lax.fori_loop(0, _NV, inner, (acc, izeros, iota),
                                  unroll=8)
        return acc

    def step(jj, acc):
        j0 = 2 * jj
        pltpu.make_async_copy(planes_hbm.at[b, c0], pl0_v, sem0).wait()
        pltpu.async_copy(planes_hbm.at[b, c0 + j0 + 1], pl1_v, sem1)
        acc = plane_acc(pl0_v, j0, acc)
        pltpu.make_async_copy(planes_hbm.at[b, c0], pl1_v, sem1).wait()

        @pl.when(jj < _CPT // 2 - 1)
        def _():
            pltpu.async_copy(planes_hbm.at[b, c0 + j0 + 2], pl0_v, sem0)

        return plane_acc(pl1_v, j0 + 1, acc)

    acc = lax.fori_loop(0, _CPT // 2, step, zeros)

    def msum_step(i, a):
        return a + m_v[pl.ds(i * _L, _L)]

    msum = lax.fori_loop(0, _NV, msum_step, zeros)
    msum = msum * (half == 0).astype(jnp.float32)

    red_v[0, :] = acc
    red_v[1, :] = msum
    pltpu.sync_copy(red_v, out_hbm.at[cid, sid])


@functools.cache
def _sc_kernel():
    return functools.partial(
        pl.kernel,
        out_type=jax.ShapeDtypeStruct((_NC, _NS, 2, _L), jnp.float32),
        mesh=plsc.VectorSubcoreMesh(
            core_axis_name="c", subcore_axis_name="s",
            num_cores=_NC, num_subcores=_NS),
        compiler_params=pltpu.CompilerParams(
            needs_layout_passes=False, use_tc_tiling_on_sc=False),
        scratch_types=[
            pltpu.VMEM((_N,), jnp.int32),
            pltpu.VMEM((_N,), jnp.float32),
            pltpu.VMEM((_MO, _MP, _CPT), jnp.float32),
            pltpu.VMEM((_H, _W), jnp.float32),
            pltpu.VMEM((_H, _W), jnp.float32),
            pltpu.VMEM((2, _L), jnp.float32),
            pltpu.SemaphoreType.DMA,
            pltpu.SemaphoreType.DMA,
        ],
    )(_sc_body)


def kernel(output, target, ind, ind_mask):
    b, C, H, W = output.shape
    parts = _sc_kernel()(output, target,
                         ind.reshape(b, _N), ind_mask.reshape(b, _N))
    abs_sum = jnp.sum(parts[:, :, 0, :])
    mask_sum = jnp.sum(parts[:, :, 1, :])
    return abs_sum / (C * mask_sum + 0.0001)
